# baseline (speedup 1.0000x reference)
import functools

import jax
import jax.numpy as jnp
from jax import lax
from jax.experimental import pallas as pl
from jax.experimental.pallas import tpu as pltpu
from jax.experimental.pallas import tpu_sc as plsc

_Q = 254
_K = 256
_D = 512
_V = 509

_NC = 2
_NS = 16

_QPW = 16
_CHUNK = 64
_NCHUNK = _K // _CHUNK
_KT = _CHUNK // 8
_DT = _D // 128
_DH = _DT // _NC
_TT = _K // 8 + _QPW - 1


_G0 = 24


def _body(w_hbm, out_hbm, tbl, ssem, ssem1, wsem):
    half = lax.axis_index("c")
    sid = lax.axis_index("s")
    r = sid & 7
    i0 = (sid >> 3) * _QPW
    qmax = r + 8 * (i0 + _QPW - 1)
    qmaxc = jnp.where(qmax >= _Q, qmax - 8, qmax)
    base = (_Q - 1) - qmaxc

    def stage(lo, hi, sem):
        def body(tt, carry):
            for g in range(_DH):
                pltpu.async_copy(
                    w_hbm.at[pl.ds(base + 8 * tt, 8), _DH * half + g, :],
                    tbl.at[tt, g],
                    sem,
                )
            return carry

        lax.fori_loop(lo, hi, body, 0)

    stage(0, _G0, ssem)
    stage(_G0, _TT, ssem1)

    def write_chunk(i, c):
        q = r + 8 * (i0 + i)
        qc = jnp.where(q >= _Q, q - 8, q)
        jt = _KT * c + ((qmaxc - qc) >> 3)
        pltpu.async_copy(
            tbl.at[pl.ds(jt, _KT)],
            out_hbm.at[qc, pl.ds(c * _KT, _KT), pl.ds(_DH * half, _DH), :, :],
            wsem,
        )

    pltpu.make_async_copy(
        out_hbm.at[0, pl.ds(0, _G0), pl.ds(0, _DH), :, :],
        tbl.at[pl.ds(0, _G0)],
        ssem,
    ).wait()
    lax.fori_loop(0, _QPW, lambda i, cr: (write_chunk(i, 0), cr)[1], 0)
    pltpu.make_async_copy(
        out_hbm.at[0, pl.ds(0, _TT - _G0), pl.ds(0, _DH), :, :],
        tbl.at[pl.ds(_G0, _TT - _G0)],
        ssem1,
    ).wait()

    def write_rest(i, carry):
        for c in range(1, _NCHUNK):
            write_chunk(i, c)
        return carry

    lax.fori_loop(0, _QPW, write_rest, 0)
    for hq in range(2):
        pltpu.make_async_copy(
            out_hbm.at[hq, :, pl.ds(0, _DH), :, :],
            out_hbm.at[hq, :, pl.ds(0, _DH), :, :],
            wsem,
        ).wait()


@jax.jit
def kernel(weight):
    run = functools.partial(
        pl.kernel,
        out_type=jax.ShapeDtypeStruct((_Q, _K // 8, _DT, 8, 128), jnp.float32),
        mesh=plsc.VectorSubcoreMesh(core_axis_name="c", subcore_axis_name="s"),
        scratch_types=[
            pltpu.VMEM((_TT, _DH, 8, 128), jnp.float32),
            pltpu.SemaphoreType.DMA,
            pltpu.SemaphoreType.DMA,
            pltpu.SemaphoreType.DMA,
        ],
        compiler_params=pltpu.CompilerParams(use_tc_tiling_on_sc=False),
    )(_body)
    tiled = run(weight.reshape(_V, _DT, 128))
    return tiled.transpose(0, 1, 3, 2, 4).reshape(_Q, _K, _D)


# speedup vs baseline: 1.0065x; 1.0065x over previous
import functools

import jax
import jax.numpy as jnp
from jax import lax
from jax.experimental import pallas as pl
from jax.experimental.pallas import tpu as pltpu
from jax.experimental.pallas import tpu_sc as plsc

_Q = 254
_K = 256
_D = 512
_V = 509

_NC = 2
_NS = 16

_QPW = 16
_CHUNK = 64
_NCHUNK = _K // _CHUNK
_KT = _CHUNK // 8
_DT = _D // 128
_DH = _DT // _NC
_TT = _K // 8 + _QPW - 1


def _body(w_hbm, out_hbm, tbl, ssem, wsem):
    half = lax.axis_index("c")
    sid = lax.axis_index("s")
    r = sid & 7
    i0 = (sid >> 3) * _QPW
    qmax = r + 8 * (i0 + _QPW - 1)
    qmaxc = jnp.where(qmax >= _Q, qmax - 8, qmax)
    base = (_Q - 1) - qmaxc

    def stage(tt, carry):
        for g in range(_DH):
            pltpu.async_copy(
                w_hbm.at[pl.ds(base + 8 * tt, 8), _DH * half + g, :],
                tbl.at[tt, g],
                ssem,
            )
        return carry

    lax.fori_loop(0, _TT, stage, 0)
    pltpu.make_async_copy(
        out_hbm.at[0, pl.ds(0, _TT), pl.ds(0, _DH), :, :], tbl, ssem
    ).wait()

    def write_row(i, carry):
        q = r + 8 * (i0 + i)
        qc = jnp.where(q >= _Q, q - 8, q)
        for c in range(_NCHUNK):
            jt = _KT * c + ((qmaxc - qc) >> 3)
            pltpu.async_copy(
                tbl.at[pl.ds(jt, _KT)],
                out_hbm.at[qc, pl.ds(c * _KT, _KT), pl.ds(_DH * half, _DH), :, :],
                wsem,
            )
        return carry

    lax.fori_loop(0, _QPW, write_row, 0)
    for hq in range(2):
        pltpu.make_async_copy(
            out_hbm.at[hq, :, pl.ds(0, _DH), :, :],
            out_hbm.at[hq, :, pl.ds(0, _DH), :, :],
            wsem,
        ).wait()


@jax.jit
def kernel(weight):
    run = functools.partial(
        pl.kernel,
        out_type=jax.ShapeDtypeStruct((_Q, _K // 8, _DT, 8, 128), jnp.float32),
        mesh=plsc.VectorSubcoreMesh(core_axis_name="c", subcore_axis_name="s"),
        scratch_types=[
            pltpu.VMEM((_TT, _DH, 8, 128), jnp.float32),
            pltpu.SemaphoreType.DMA,
            pltpu.SemaphoreType.DMA,
        ],
        compiler_params=pltpu.CompilerParams(use_tc_tiling_on_sc=False),
    )(_body)
    tiled = run(weight.reshape(_V, _DT, 128))
    return tiled.transpose(0, 1, 3, 2, 4).reshape(_Q, _K, _D)
